# baseline (speedup 1.0000x reference)
import jax
import jax.numpy as jnp
from jax.experimental import pallas as pl

B, MAX_ATOMS, MAX_DEGREE = 512, 100, 6
NUM_ATOM_FEATURES, NUM_BOND_FEATURES, CONV_WIDTH = 128, 16, 128
BM = 8


def _body(atoms_ref, bonds_ref, edges_ref, W_ref, b_ref, out_ref):
    e = edges_ref[...]

    bsel_i = jax.lax.broadcasted_iota(jnp.int32, (MAX_DEGREE * NUM_BOND_FEATURES, NUM_BOND_FEATURES), 0)
    bsel_j = jax.lax.broadcasted_iota(jnp.int32, (MAX_DEGREE * NUM_BOND_FEATURES, NUM_BOND_FEATURES), 1)
    bsel = jnp.where(bsel_i % NUM_BOND_FEATURES == bsel_j, 1.0, 0.0)
    s_bond = jax.lax.dot(bonds_ref[...], bsel, preferred_element_type=jnp.float32)

    lane = jax.lax.broadcasted_iota(jnp.int32, (MAX_ATOMS, MAX_ATOMS), 1)
    for m in range(BM):
        e_m = e[m]
        at_m = atoms_ref[m]
        amat = jnp.zeros((MAX_ATOMS, MAX_ATOMS), dtype=jnp.float32)
        for d in range(MAX_DEGREE):
            amat = amat + jnp.where(e_m[:, d:d + 1] == lane, 1.0, 0.0)
        s_atom_m = at_m + jax.lax.dot(amat, at_m, preferred_element_type=jnp.float32)
        s_atom_bf = s_atom_m.astype(jnp.bfloat16)
        s_bond_bf = s_bond[m * MAX_ATOMS:(m + 1) * MAX_ATOMS, :].astype(jnp.bfloat16)

        slot_valid = [e_m[:, d:d + 1] >= 0 for d in range(MAX_DEGREE)]
        acc = jnp.zeros((MAX_ATOMS, CONV_WIDTH), dtype=jnp.float32)
        for d in range(MAX_DEGREE):
            y = (
                jax.lax.dot(s_atom_bf, W_ref[d, :NUM_ATOM_FEATURES, :],
                            preferred_element_type=jnp.float32)
                + jax.lax.dot(s_bond_bf, W_ref[d, NUM_ATOM_FEATURES:, :],
                              preferred_element_type=jnp.float32)
                + b_ref[d][None, :]
            )
            y = jax.nn.relu(y)
            if d == 0:
                mask = ~slot_valid[0]
            else:
                mask = slot_valid[d - 1] & ~slot_valid[d]
            acc = acc + jnp.where(mask, y, 0.0)
        out_ref[m] = acc


@jax.jit
def kernel(atoms, bonds, edges, W, b):
    bonds2d = bonds.reshape(B * MAX_ATOMS, MAX_DEGREE * NUM_BOND_FEATURES)
    return pl.pallas_call(
        _body,
        grid=(B // BM,),
        in_specs=[
            pl.BlockSpec((BM, MAX_ATOMS, NUM_ATOM_FEATURES), lambda i: (i, 0, 0)),
            pl.BlockSpec((BM * MAX_ATOMS, MAX_DEGREE * NUM_BOND_FEATURES), lambda i: (i, 0)),
            pl.BlockSpec((BM, MAX_ATOMS, MAX_DEGREE), lambda i: (i, 0, 0)),
            pl.BlockSpec((MAX_DEGREE, NUM_ATOM_FEATURES + NUM_BOND_FEATURES, CONV_WIDTH),
                         lambda i: (0, 0, 0)),
            pl.BlockSpec((MAX_DEGREE, CONV_WIDTH), lambda i: (0, 0)),
        ],
        out_specs=pl.BlockSpec((BM, MAX_ATOMS, CONV_WIDTH), lambda i: (i, 0, 0)),
        out_shape=jax.ShapeDtypeStruct((B, MAX_ATOMS, CONV_WIDTH), jnp.float32),
    )(atoms, bonds2d, edges.astype(jnp.int32), W.astype(jnp.bfloat16), b)


# speedup vs baseline: 1.1291x; 1.1291x over previous
import jax
import jax.numpy as jnp
from jax.experimental import pallas as pl
from jax.experimental.pallas import tpu as pltpu

B, MAX_ATOMS, MAX_DEGREE = 512, 100, 6
NUM_ATOM_FEATURES, NUM_BOND_FEATURES, CONV_WIDTH = 128, 16, 128
BM = 8
ROWS = BM * MAX_ATOMS


def _body(atoms_ref, bonds_ref, edgesT_ref, W_ref, b_ref, out_ref,
          satom_ref, ed_ref):

    bsel_i = jax.lax.broadcasted_iota(jnp.int32, (MAX_DEGREE * NUM_BOND_FEATURES, NUM_BOND_FEATURES), 0)
    bsel_j = jax.lax.broadcasted_iota(jnp.int32, (MAX_DEGREE * NUM_BOND_FEATURES, NUM_BOND_FEATURES), 1)
    bsel = jnp.where(bsel_i % NUM_BOND_FEATURES == bsel_j, 1.0, 0.0)
    s_bond = jax.lax.dot(bonds_ref[...], bsel, preferred_element_type=jnp.float32)

    sub_iota = jax.lax.broadcasted_iota(jnp.int32, (MAX_ATOMS, MAX_ATOMS), 0)
    for m in range(BM):
        sl = slice(m * MAX_ATOMS, (m + 1) * MAX_ATOMS)
        eT = edgesT_ref[m]
        at_m = atoms_ref[sl, :]
        amat_T = jnp.zeros((MAX_ATOMS, MAX_ATOMS), dtype=jnp.float32)
        for d in range(MAX_DEGREE):
            amat_T = amat_T + jnp.where(eT[d:d + 1, :] == sub_iota, 1.0, 0.0)
        nbr = jax.lax.dot_general(amat_T, at_m, (((0,), (0,)), ((), ())),
                                  preferred_element_type=jnp.float32)
        satom_ref[sl, :] = at_m + nbr
        ed_ref[sl, :] = jnp.transpose(eT)

    x = satom_ref[...]
    ed = ed_ref[...]
    acc = jnp.zeros((ROWS, CONV_WIDTH), dtype=jnp.float32)
    for d in range(MAX_DEGREE):
        y = (
            jax.lax.dot(x, W_ref[d, :NUM_ATOM_FEATURES, :],
                        preferred_element_type=jnp.float32)
            + jax.lax.dot(s_bond, W_ref[d, NUM_ATOM_FEATURES:, :],
                          preferred_element_type=jnp.float32)
            + b_ref[d][None, :]
        )
        y = jax.nn.relu(y)
        if d == 0:
            mask = ed[:, 0:1] < 0
        else:
            mask = (ed[:, d - 1:d] >= 0) & (ed[:, d:d + 1] < 0)
        acc = acc + jnp.where(mask, y, 0.0)
    out_ref[...] = acc


@jax.jit
def kernel(atoms, bonds, edges, W, b):
    atoms2d = atoms.reshape(B * MAX_ATOMS, NUM_ATOM_FEATURES)
    bonds2d = bonds.reshape(B * MAX_ATOMS, MAX_DEGREE * NUM_BOND_FEATURES)
    edgesT = jnp.transpose(edges.astype(jnp.int32), (0, 2, 1))

    out = pl.pallas_call(
        _body,
        grid=(B // BM,),
        in_specs=[
            pl.BlockSpec((ROWS, NUM_ATOM_FEATURES), lambda i: (i, 0)),
            pl.BlockSpec((ROWS, MAX_DEGREE * NUM_BOND_FEATURES), lambda i: (i, 0)),
            pl.BlockSpec((BM, MAX_DEGREE, MAX_ATOMS), lambda i: (i, 0, 0)),
            pl.BlockSpec((MAX_DEGREE, NUM_ATOM_FEATURES + NUM_BOND_FEATURES, CONV_WIDTH),
                         lambda i: (0, 0, 0)),
            pl.BlockSpec((MAX_DEGREE, CONV_WIDTH), lambda i: (0, 0)),
        ],
        out_specs=pl.BlockSpec((ROWS, CONV_WIDTH), lambda i: (i, 0)),
        out_shape=jax.ShapeDtypeStruct((B * MAX_ATOMS, CONV_WIDTH), jnp.float32),
        scratch_shapes=[
            pltpu.VMEM((ROWS, NUM_ATOM_FEATURES), jnp.float32),
            pltpu.VMEM((ROWS, MAX_DEGREE), jnp.int32),
        ],
    )(atoms2d, bonds2d, edgesT, W, b)
    return out.reshape(B, MAX_ATOMS, CONV_WIDTH)


# speedup vs baseline: 1.3769x; 1.2195x over previous
import jax
import jax.numpy as jnp
from jax.experimental import pallas as pl
from jax.experimental.pallas import tpu as pltpu

B, MAX_ATOMS, MAX_DEGREE = 512, 100, 6
NUM_ATOM_FEATURES, NUM_BOND_FEATURES, CONV_WIDTH = 128, 16, 128
BM = 8
ROWS = BM * MAX_ATOMS


def _body(atoms_ref, bonds_ref, edgesT_ref, W_ref, b_ref, out_ref,
          satom_ref, ed_ref):

    bsel_i = jax.lax.broadcasted_iota(jnp.int32, (MAX_DEGREE * NUM_BOND_FEATURES, NUM_BOND_FEATURES), 0)
    bsel_j = jax.lax.broadcasted_iota(jnp.int32, (MAX_DEGREE * NUM_BOND_FEATURES, NUM_BOND_FEATURES), 1)
    bsel = jnp.where(bsel_i % NUM_BOND_FEATURES == bsel_j, 1.0, 0.0)
    s_bond = jax.lax.dot(bonds_ref[...], bsel, preferred_element_type=jnp.float32)

    sub_iota = jax.lax.broadcasted_iota(jnp.int32, (MAX_ATOMS, MAX_ATOMS), 0)
    for m in range(BM):
        sl = slice(m * MAX_ATOMS, (m + 1) * MAX_ATOMS)
        eT = edgesT_ref[m]
        at_m = atoms_ref[m]
        amat_T = jnp.zeros((MAX_ATOMS, MAX_ATOMS), dtype=jnp.float32)
        for d in range(MAX_DEGREE):
            amat_T = amat_T + jnp.where(eT[d:d + 1, :] == sub_iota, 1.0, 0.0)
        nbr = jax.lax.dot_general(amat_T, at_m, (((0,), (0,)), ((), ())),
                                  preferred_element_type=jnp.float32)
        satom_ref[sl, :] = at_m + nbr
        ed_ref[sl, :] = jnp.transpose(eT)

    x = satom_ref[...]
    ed = ed_ref[...]
    acc = jnp.zeros((ROWS, CONV_WIDTH), dtype=jnp.float32)
    for d in range(MAX_DEGREE):
        y = (
            jax.lax.dot(x, W_ref[d, :NUM_ATOM_FEATURES, :],
                        preferred_element_type=jnp.float32)
            + jax.lax.dot(s_bond, W_ref[d, NUM_ATOM_FEATURES:, :],
                          preferred_element_type=jnp.float32)
            + b_ref[d][None, :]
        )
        y = jax.nn.relu(y)
        if d == 0:
            mask = ed[:, 0:1] < 0
        else:
            mask = (ed[:, d - 1:d] >= 0) & (ed[:, d:d + 1] < 0)
        acc = acc + jnp.where(mask, y, 0.0)
    for m in range(BM):
        out_ref[m] = acc[m * MAX_ATOMS:(m + 1) * MAX_ATOMS, :]


@jax.jit
def kernel(atoms, bonds, edges, W, b):
    bonds2d = bonds.reshape(B * MAX_ATOMS, MAX_DEGREE * NUM_BOND_FEATURES)
    edgesT = jnp.transpose(edges.astype(jnp.int32), (0, 2, 1))

    out = pl.pallas_call(
        _body,
        grid=(B // BM,),
        in_specs=[
            pl.BlockSpec((BM, MAX_ATOMS, NUM_ATOM_FEATURES), lambda i: (i, 0, 0)),
            pl.BlockSpec((ROWS, MAX_DEGREE * NUM_BOND_FEATURES), lambda i: (i, 0)),
            pl.BlockSpec((BM, MAX_DEGREE, MAX_ATOMS), lambda i: (i, 0, 0)),
            pl.BlockSpec((MAX_DEGREE, NUM_ATOM_FEATURES + NUM_BOND_FEATURES, CONV_WIDTH),
                         lambda i: (0, 0, 0)),
            pl.BlockSpec((MAX_DEGREE, CONV_WIDTH), lambda i: (0, 0)),
        ],
        out_specs=pl.BlockSpec((BM, MAX_ATOMS, CONV_WIDTH), lambda i: (i, 0, 0)),
        out_shape=jax.ShapeDtypeStruct((B, MAX_ATOMS, CONV_WIDTH), jnp.float32),
        scratch_shapes=[
            pltpu.VMEM((ROWS, NUM_ATOM_FEATURES), jnp.float32),
            pltpu.VMEM((ROWS, MAX_DEGREE), jnp.int32),
        ],
    )(atoms, bonds2d, edgesT, W, b)
    return out


# speedup vs baseline: 1.3835x; 1.0048x over previous
import jax
import jax.numpy as jnp
from jax.experimental import pallas as pl
from jax.experimental.pallas import tpu as pltpu

B, MAX_ATOMS, MAX_DEGREE = 512, 100, 6
NUM_ATOM_FEATURES, NUM_BOND_FEATURES, CONV_WIDTH = 128, 16, 128
BM = 8
ROWS = BM * MAX_ATOMS


def _body(atoms_ref, bonds_ref, edgesT_ref, W_ref, b_ref, out_ref,
          satom_ref, ed_ref):

    bsel_i = jax.lax.broadcasted_iota(jnp.int32, (MAX_DEGREE * NUM_BOND_FEATURES, NUM_BOND_FEATURES), 0)
    bsel_j = jax.lax.broadcasted_iota(jnp.int32, (MAX_DEGREE * NUM_BOND_FEATURES, NUM_BOND_FEATURES), 1)
    bsel = jnp.where(bsel_i % NUM_BOND_FEATURES == bsel_j, 1.0, 0.0)
    s_bond = jax.lax.dot(bonds_ref[...], bsel, preferred_element_type=jnp.float32)

    sub_iota = jax.lax.broadcasted_iota(jnp.int32, (MAX_ATOMS, MAX_ATOMS), 0)
    for m in range(BM):
        sl = slice(m * MAX_ATOMS, (m + 1) * MAX_ATOMS)
        eT = edgesT_ref[m]
        at_m = atoms_ref[m]
        amat_T = jnp.zeros((MAX_ATOMS, MAX_ATOMS), dtype=jnp.float32)
        for d in range(MAX_DEGREE):
            amat_T = amat_T + jnp.where(eT[d:d + 1, :] == sub_iota, 1.0, 0.0)
        nbr = jax.lax.dot_general(amat_T, at_m, (((0,), (0,)), ((), ())),
                                  preferred_element_type=jnp.float32)
        satom_ref[sl, :] = at_m + nbr
        ed_ref[sl, :] = jnp.transpose(eT)

    x = satom_ref[...].astype(jnp.bfloat16)
    s_bond = s_bond.astype(jnp.bfloat16)
    ed = ed_ref[...]
    acc = jnp.zeros((ROWS, CONV_WIDTH), dtype=jnp.float32)
    for d in range(MAX_DEGREE):
        y = (
            jax.lax.dot(x, W_ref[d, :NUM_ATOM_FEATURES, :],
                        preferred_element_type=jnp.float32)
            + jax.lax.dot(s_bond, W_ref[d, NUM_ATOM_FEATURES:, :],
                          preferred_element_type=jnp.float32)
            + b_ref[d][None, :]
        )
        y = jax.nn.relu(y)
        if d == 0:
            mask = ed[:, 0:1] < 0
        else:
            mask = (ed[:, d - 1:d] >= 0) & (ed[:, d:d + 1] < 0)
        acc = acc + jnp.where(mask, y, 0.0)
    for m in range(BM):
        out_ref[m] = acc[m * MAX_ATOMS:(m + 1) * MAX_ATOMS, :]


@jax.jit
def kernel(atoms, bonds, edges, W, b):
    bonds2d = bonds.reshape(B * MAX_ATOMS, MAX_DEGREE * NUM_BOND_FEATURES)
    edgesT = jnp.transpose(edges.astype(jnp.int32), (0, 2, 1))

    out = pl.pallas_call(
        _body,
        grid=(B // BM,),
        in_specs=[
            pl.BlockSpec((BM, MAX_ATOMS, NUM_ATOM_FEATURES), lambda i: (i, 0, 0)),
            pl.BlockSpec((ROWS, MAX_DEGREE * NUM_BOND_FEATURES), lambda i: (i, 0)),
            pl.BlockSpec((BM, MAX_DEGREE, MAX_ATOMS), lambda i: (i, 0, 0)),
            pl.BlockSpec((MAX_DEGREE, NUM_ATOM_FEATURES + NUM_BOND_FEATURES, CONV_WIDTH),
                         lambda i: (0, 0, 0)),
            pl.BlockSpec((MAX_DEGREE, CONV_WIDTH), lambda i: (0, 0)),
        ],
        out_specs=pl.BlockSpec((BM, MAX_ATOMS, CONV_WIDTH), lambda i: (i, 0, 0)),
        out_shape=jax.ShapeDtypeStruct((B, MAX_ATOMS, CONV_WIDTH), jnp.float32),
        scratch_shapes=[
            pltpu.VMEM((ROWS, NUM_ATOM_FEATURES), jnp.float32),
            pltpu.VMEM((ROWS, MAX_DEGREE), jnp.int32),
        ],
    )(atoms, bonds2d, edgesT, W.astype(jnp.bfloat16), b)
    return out
